# baseline (speedup 1.0000x reference)
import jax
import jax.numpy as jnp
from jax import lax
from jax.experimental import pallas as pl
from jax.experimental.pallas import tpu as pltpu
from jax.experimental.pallas import tpu_sc as plsc

B = 16384
ED = 64
NC = 2
NS = 16
NW = NC * NS
BPW = B // NW
CHUNK = 128
NCHUNK = BPW // CHUNK


def _user_gather_body(idx_hbm, table_hbm, out_hbm, idx_v, rows_v, sem):
    wid = lax.axis_index("s") * NC + lax.axis_index("c")
    row_base = wid * NCHUNK
    base = wid * BPW
    pltpu.sync_copy(idx_hbm.at[pl.ds(row_base, NCHUNK)], idx_v)
    descs = []
    for j in range(NCHUNK):
        descs.append(
            pltpu.async_copy(
                table_hbm.at[idx_v.at[j]],
                rows_v.at[pl.ds(j * CHUNK, CHUNK)],
                sem,
            )
        )
    for d in descs:
        d.wait()
    pltpu.sync_copy(rows_v, out_hbm.at[pl.ds(base, BPW)])


def _user_gather(idx2d, user_table):
    mesh = plsc.VectorSubcoreMesh(core_axis_name="c", subcore_axis_name="s")
    return pl.kernel(
        _user_gather_body,
        mesh=mesh,
        out_type=jax.ShapeDtypeStruct((B, ED), jnp.float32),
        scratch_types=[
            pltpu.VMEM((NCHUNK, CHUNK), jnp.int32),
            pltpu.VMEM((BPW, ED), jnp.float32),
            pltpu.SemaphoreType.DMA,
        ],
        compiler_params=pltpu.CompilerParams(use_tc_tiling_on_sc=False),
    )(idx2d, user_table)


BLK = 2048
AGE_PAD = 104
GEN_PAD = 8


def _mlp_body(ue_ref, age_ref, gen_ref, at_ref, gt_ref,
              w1_ref, b1_ref, w2_ref, b2_ref, w3_ref, b3_ref, o_ref):
    f32 = jnp.float32
    ue = ue_ref[...]
    age = age_ref[...]
    gen = gen_ref[...]

    a_iota = lax.broadcasted_iota(jnp.int32, (BLK, AGE_PAD), 1)
    aoh = (age == a_iota).astype(f32)
    ae = jnp.dot(aoh, at_ref[...], preferred_element_type=f32)

    g_iota = lax.broadcasted_iota(jnp.int32, (BLK, GEN_PAD), 1)
    goh = (gen == g_iota).astype(f32)
    ge = jnp.dot(goh, gt_ref[...], preferred_element_type=f32)

    h = (jnp.dot(ue, w1_ref[0:64, :], preferred_element_type=f32)
         + jnp.dot(ae, w1_ref[64:96, :], preferred_element_type=f32)
         + jnp.dot(ge, w1_ref[96:112, :], preferred_element_type=f32)
         + b1_ref[...])
    h = jnp.maximum(h, 0.0)
    h = jnp.maximum(jnp.dot(h, w2_ref[...], preferred_element_type=f32)
                    + b2_ref[...], 0.0)
    v = jnp.dot(h, w3_ref[...], preferred_element_type=f32) + b3_ref[...]
    ss = jnp.sum(v * v, axis=1, keepdims=True)
    o_ref[...] = v / jnp.maximum(jnp.sqrt(ss), 1e-12)


def _mlp(ue, age, gen, at_pad, gt_pad, W1, b1, W2, b2, W3, b3):
    grid = (B // BLK,)
    const = lambda i: (0, 0)
    return pl.pallas_call(
        _mlp_body,
        grid=grid,
        in_specs=[
            pl.BlockSpec((BLK, ED), lambda i: (i, 0)),
            pl.BlockSpec((BLK, 1), lambda i: (i, 0)),
            pl.BlockSpec((BLK, 1), lambda i: (i, 0)),
            pl.BlockSpec((AGE_PAD, 32), const),
            pl.BlockSpec((GEN_PAD, 16), const),
            pl.BlockSpec((112, 128), const),
            pl.BlockSpec((1, 128), const),
            pl.BlockSpec((128, 64), const),
            pl.BlockSpec((1, 64), const),
            pl.BlockSpec((64, 64), const),
            pl.BlockSpec((1, 64), const),
        ],
        out_specs=pl.BlockSpec((BLK, 64), lambda i: (i, 0)),
        out_shape=jax.ShapeDtypeStruct((B, 64), jnp.float32),
    )(ue, age, gen, at_pad, gt_pad, W1, b1, W2, b2, W3, b3)


def kernel(user_id, user_age, user_gender, user_table, age_table, gender_table,
           W1, b1, W2, b2, W3, b3):
    idx2d = user_id.reshape(B // CHUNK, CHUNK)
    ue = _user_gather(idx2d, user_table)
    at_pad = jnp.pad(age_table, ((0, AGE_PAD - age_table.shape[0]), (0, 0)))
    gt_pad = jnp.pad(gender_table, ((0, GEN_PAD - gender_table.shape[0]), (0, 0)))
    return _mlp(ue, user_age, user_gender, at_pad, gt_pad,
                W1, b1.reshape(1, -1), W2, b2.reshape(1, -1),
                W3, b3.reshape(1, -1))


# speedup vs baseline: 1.0443x; 1.0443x over previous
import jax
import jax.numpy as jnp
from jax import lax
from jax.experimental import pallas as pl
from jax.experimental.pallas import tpu as pltpu
from jax.experimental.pallas import tpu_sc as plsc

B = 16384
ED = 64
NC = 2
NS = 16
NW = NC * NS
BPW = B // NW
CHUNK = 128
NCHUNK = BPW // CHUNK


LANES = 16


def _user_gather_body(idx_hbm, table_hbm, out_hbm, idx_v, sem):
    wid = lax.axis_index("s") * NC + lax.axis_index("c")
    base = wid * BPW
    pltpu.sync_copy(idx_hbm.at[pl.ds(base, BPW)], idx_v)
    lane = lax.broadcasted_iota(jnp.int32, (LANES,), 0)

    def fire_group(g, c):
        vec = idx_v[pl.ds(g * LANES, LANES)]
        for j in range(LANES):
            idx = jnp.sum(jnp.where(lane == j, vec, 0))
            pltpu.make_async_copy(
                table_hbm.at[pl.ds(idx, 1)],
                out_hbm.at[pl.ds(base + g * LANES + j, 1)],
                sem,
            ).start()
        return c

    lax.fori_loop(0, BPW // LANES, fire_group, 0)
    pltpu.make_async_copy(
        table_hbm.at[pl.ds(0, BPW)],
        out_hbm.at[pl.ds(base, BPW)],
        sem,
    ).wait()


def _user_gather(idx1d, user_table):
    mesh = plsc.VectorSubcoreMesh(core_axis_name="c", subcore_axis_name="s")
    return pl.kernel(
        _user_gather_body,
        mesh=mesh,
        out_type=jax.ShapeDtypeStruct((B, ED), jnp.float32),
        scratch_types=[
            pltpu.VMEM((BPW,), jnp.int32),
            pltpu.SemaphoreType.DMA,
        ],
        compiler_params=pltpu.CompilerParams(needs_layout_passes=False),
    )(idx1d, user_table)


BLK = 2048
AGE_PAD = 104
GEN_PAD = 8


def _mlp_body(ue_ref, age_ref, gen_ref, at_ref, gt_ref,
              w1_ref, b1_ref, w2_ref, b2_ref, w3_ref, b3_ref, o_ref):
    f32 = jnp.float32
    ue = ue_ref[...]
    age = age_ref[...]
    gen = gen_ref[...]

    a_iota = lax.broadcasted_iota(jnp.int32, (BLK, AGE_PAD), 1)
    aoh = (age == a_iota).astype(f32)
    ae = jnp.dot(aoh, at_ref[...], preferred_element_type=f32)

    g_iota = lax.broadcasted_iota(jnp.int32, (BLK, GEN_PAD), 1)
    goh = (gen == g_iota).astype(f32)
    ge = jnp.dot(goh, gt_ref[...], preferred_element_type=f32)

    h = (jnp.dot(ue, w1_ref[0:64, :], preferred_element_type=f32)
         + jnp.dot(ae, w1_ref[64:96, :], preferred_element_type=f32)
         + jnp.dot(ge, w1_ref[96:112, :], preferred_element_type=f32)
         + b1_ref[...])
    h = jnp.maximum(h, 0.0)
    h = jnp.maximum(jnp.dot(h, w2_ref[...], preferred_element_type=f32)
                    + b2_ref[...], 0.0)
    v = jnp.dot(h, w3_ref[...], preferred_element_type=f32) + b3_ref[...]
    ss = jnp.sum(v * v, axis=1, keepdims=True)
    o_ref[...] = v / jnp.maximum(jnp.sqrt(ss), 1e-12)


def _mlp(ue, age, gen, at_pad, gt_pad, W1, b1, W2, b2, W3, b3):
    grid = (B // BLK,)
    const = lambda i: (0, 0)
    return pl.pallas_call(
        _mlp_body,
        grid=grid,
        in_specs=[
            pl.BlockSpec((BLK, ED), lambda i: (i, 0)),
            pl.BlockSpec((BLK, 1), lambda i: (i, 0)),
            pl.BlockSpec((BLK, 1), lambda i: (i, 0)),
            pl.BlockSpec((AGE_PAD, 32), const),
            pl.BlockSpec((GEN_PAD, 16), const),
            pl.BlockSpec((112, 128), const),
            pl.BlockSpec((1, 128), const),
            pl.BlockSpec((128, 64), const),
            pl.BlockSpec((1, 64), const),
            pl.BlockSpec((64, 64), const),
            pl.BlockSpec((1, 64), const),
        ],
        out_specs=pl.BlockSpec((BLK, 64), lambda i: (i, 0)),
        out_shape=jax.ShapeDtypeStruct((B, 64), jnp.float32),
    )(ue, age, gen, at_pad, gt_pad, W1, b1, W2, b2, W3, b3)


def kernel(user_id, user_age, user_gender, user_table, age_table, gender_table,
           W1, b1, W2, b2, W3, b3):
    ue = _user_gather(user_id.reshape(B), user_table)
    at_pad = jnp.pad(age_table, ((0, AGE_PAD - age_table.shape[0]), (0, 0)))
    gt_pad = jnp.pad(gender_table, ((0, GEN_PAD - gender_table.shape[0]), (0, 0)))
    return _mlp(ue, user_age, user_gender, at_pad, gt_pad,
                W1, b1.reshape(1, -1), W2, b2.reshape(1, -1),
                W3, b3.reshape(1, -1))
